# baseline (speedup 1.0000x reference)
import functools

import jax
import jax.numpy as jnp
from jax import lax
from jax.experimental import pallas as pl
from jax.experimental.pallas import tpu as pltpu
from jax.experimental.pallas import tpu_sc as plsc

N = 10000
E = 320000
D = 128
G = 128

NC = 2
NS = 16
NW = NC * NS
K = 128
NP = 10240
RPT = NP // NW
SROWS = NS * RPT + 256
CAP = 11264
C2 = CAP // K

CAPG = 512
GPT = G // NW
CG2 = CAPG // K


def _sc_segment_sum(h, src4, dst4, zrows):
    mesh = plsc.VectorSubcoreMesh(core_axis_name="c", subcore_axis_name="s")

    @functools.partial(
        pl.kernel,
        out_type=jax.ShapeDtypeStruct((NP, D), jnp.float32),
        mesh=mesh,
        scratch_types=[
            pltpu.VMEM((C2, K), jnp.int32),
            pltpu.VMEM((C2, K), jnp.int32),
            pltpu.VMEM((K, D), jnp.float32),
            pltpu.VMEM((K, D), jnp.float32),
            pltpu.VMEM_SHARED((SROWS, D), jnp.float32),
            pltpu.SemaphoreType.DMA,
            pltpu.SemaphoreType.DMA,
        ],
    )
    def seg_sum(h_hbm, src_hbm, dst_hbm, z_hbm, out_hbm,
                src_v, dst_v, buf0, buf1, agg_sh, sem0, sem1):
        c = lax.axis_index("c")
        s = lax.axis_index("s")

        pltpu.sync_copy(z_hbm, agg_sh.at[pl.ds(s * RPT, RPT)])
        pltpu.sync_copy(src_hbm.at[c, s], src_v)
        pltpu.sync_copy(dst_hbm.at[c, s], dst_v)
        plsc.subcore_barrier()

        def body(i, carry):
            c0 = 2 * i
            c1 = 2 * i + 1
            d0 = pltpu.async_copy(h_hbm.at[src_v.at[c0]], buf0, sem0)
            d1 = pltpu.async_copy(h_hbm.at[src_v.at[c1]], buf1, sem1)
            d0.wait()
            pltpu.sync_copy(buf0, agg_sh.at[dst_v.at[c0]], add=True)
            d1.wait()
            pltpu.sync_copy(buf1, agg_sh.at[dst_v.at[c1]], add=True)
            return carry
        lax.fori_loop(0, C2 // 2, body, 0)
        plsc.subcore_barrier()

        pltpu.sync_copy(agg_sh.at[pl.ds(s * RPT, RPT)],
                        out_hbm.at[pl.ds((c * NS + s) * RPT, RPT)])

    return seg_sum(h, src4, dst4, zrows)


def _sc_pool(h, row4, gdst4, zrows):
    mesh = plsc.VectorSubcoreMesh(core_axis_name="c", subcore_axis_name="s")
    BLK = 2 * GPT

    @functools.partial(
        pl.kernel,
        out_type=jax.ShapeDtypeStruct((NW, BLK, D), jnp.float32),
        mesh=mesh,
        scratch_types=[
            pltpu.VMEM((CG2, K), jnp.int32),
            pltpu.VMEM((CG2, K), jnp.int32),
            pltpu.VMEM((K, D), jnp.float32),
            pltpu.VMEM((K, D), jnp.float32),
            pltpu.VMEM_SHARED((NS * BLK, D), jnp.float32),
            pltpu.SemaphoreType.DMA,
            pltpu.SemaphoreType.DMA,
        ],
    )
    def pool(h_hbm, row_hbm, gdst_hbm, z_hbm, out_hbm,
             row_v, gdst_v, buf0, buf1, agg_sh, sem0, sem1):
        c = lax.axis_index("c")
        s = lax.axis_index("s")

        pltpu.sync_copy(z_hbm.at[pl.ds(0, BLK)],
                        agg_sh.at[pl.ds(s * BLK, BLK)])
        pltpu.sync_copy(row_hbm.at[c, s], row_v)
        pltpu.sync_copy(gdst_hbm.at[c, s], gdst_v)
        plsc.subcore_barrier()

        def body(i, carry):
            c0 = 2 * i
            c1 = 2 * i + 1
            d0 = pltpu.async_copy(h_hbm.at[row_v.at[c0]], buf0, sem0)
            d1 = pltpu.async_copy(h_hbm.at[row_v.at[c1]], buf1, sem1)
            d0.wait()
            pltpu.sync_copy(buf0, agg_sh.at[gdst_v.at[c0]], add=True)
            d1.wait()
            pltpu.sync_copy(buf1, agg_sh.at[gdst_v.at[c1]], add=True)
            return carry
        lax.fori_loop(0, CG2 // 2, body, 0)
        plsc.subcore_barrier()

        pltpu.sync_copy(agg_sh.at[pl.ds(s * BLK, BLK)],
                        out_hbm.at[c * NS + s])

    return pool(h, row4, gdst4, zrows)


def _mm_gin(h, agg, wt, b, e11):
    def body(h_ref, a_ref, w_ref, b_ref, e_ref, o_ref):
        xb = e_ref[0, 0] * h_ref[...] + a_ref[:N]
        o_ref[...] = jnp.dot(
            xb, w_ref[...], preferred_element_type=jnp.float32) + b_ref[...]

    return pl.pallas_call(
        body,
        out_shape=jax.ShapeDtypeStruct((N, wt.shape[1]), jnp.float32),
    )(h, agg, wt, b, e11)


def _mm_bias(x, wt, b):
    def body(x_ref, w_ref, b_ref, o_ref):
        o_ref[...] = jnp.dot(
            x_ref[...], w_ref[...],
            preferred_element_type=jnp.float32) + b_ref[...]

    return pl.pallas_call(
        body,
        out_shape=jax.ShapeDtypeStruct((x.shape[0], wt.shape[1]), jnp.float32),
    )(x, wt, b)


def _bn_act(x, g, b):
    m = jnp.mean(x, axis=0)
    v = jnp.var(x, axis=0)
    return jax.nn.relu(g * (x - m) / jnp.sqrt(v + 1e-5) + b)


def kernel(x, edge_index, batch, params, mlp_params):
    src = edge_index[0]
    dst = edge_index[1]

    owner = dst // RPT
    perm = jnp.argsort(owner, stable=True)
    srcp = src[perm]
    dstp = dst[perm]
    owner_sorted = owner[perm]
    bids = jnp.arange(NW, dtype=jnp.int32)
    starts = jnp.searchsorted(owner_sorted, bids, side='left').astype(jnp.int32)
    ends = jnp.concatenate([starts[1:], jnp.array([E], jnp.int32)])
    slots = jnp.arange(CAP, dtype=jnp.int32)[None, :]
    idxm = starts[:, None] + slots
    valid = idxm < ends[:, None]
    idxc = jnp.minimum(idxm, E - 1)
    sc_of_b = (bids // NS)[:, None]
    psrc = jnp.where(valid, srcp[idxc], (idxm * 61) % N)
    pdst = jnp.where(valid, dstp[idxc] - sc_of_b * (NS * RPT),
                     NS * RPT + (idxm % 256))
    src4 = psrc.reshape(NC, NS, C2, K)
    dst4 = pdst.reshape(NC, NS, C2, K)

    gstarts = jnp.searchsorted(
        batch, (bids * GPT).astype(jnp.int32), side='left').astype(jnp.int32)
    gends = jnp.concatenate([gstarts[1:], jnp.array([N], jnp.int32)])
    gslots = jnp.arange(CAPG, dtype=jnp.int32)[None, :]
    gidx = gstarts[:, None] + gslots
    gvalid = gidx < gends[:, None]
    gidxc = jnp.minimum(gidx, N - 1)
    bloc = (bids % NS)[:, None] * (2 * GPT)
    prow = jnp.where(gvalid, gidxc, (gidx * 61) % N)
    pgdst = jnp.where(gvalid, batch[gidxc] - bids[:, None] * GPT + bloc,
                      bloc + GPT + (gidx % GPT))
    row4 = prow.reshape(NC, NS, CG2, K)
    gdst4 = pgdst.reshape(NC, NS, CG2, K)

    zrows = jnp.zeros((RPT, D), jnp.float32)

    h = x
    for p in params:
        agg = _sc_segment_sum(h, src4, dst4, zrows)
        z = _mm_gin(h, agg, p['W1'].T, p['b1'].reshape(1, -1),
                    (1.0 + p['eps']).reshape(1, 1))
        z = _bn_act(z, p['g1'], p['be1'])
        z = _mm_bias(z, p['W2'].T, p['b2'].reshape(1, -1))
        h = _bn_act(z, p['g2'], p['be2'])

    pooled = _sc_pool(h, row4, gdst4, zrows)[:, :GPT, :].reshape(G, D)
    z = _mm_bias(pooled, mlp_params['W1'].T, mlp_params['b1'].reshape(1, -1))
    z = _bn_act(z, mlp_params['g'], mlp_params['be'])
    return _mm_bias(z, mlp_params['W2'].T, mlp_params['b2'].reshape(1, -1))


# speedup vs baseline: 1.0869x; 1.0869x over previous
import functools

import jax
import jax.numpy as jnp
from jax import lax
from jax.experimental import pallas as pl
from jax.experimental.pallas import tpu as pltpu
from jax.experimental.pallas import tpu_sc as plsc

N = 10000
E = 320000
D = 128
G = 128

NC = 2
NS = 16
NW = NC * NS
K = 128
NP = 10240
RPT = NP // NW
SROWS = NS * RPT + 256
CAP = 10752
C2 = CAP // K

CAPG = 512
GPT = G // NW
CG2 = CAPG // K


def _sc_segment_sum(h, src4, dst4, zrows):
    mesh = plsc.VectorSubcoreMesh(core_axis_name="c", subcore_axis_name="s")

    @functools.partial(
        pl.kernel,
        out_type=jax.ShapeDtypeStruct((NP, D), jnp.float32),
        mesh=mesh,
        scratch_types=[
            pltpu.VMEM((C2, K), jnp.int32),
            pltpu.VMEM((C2, K), jnp.int32),
            pltpu.VMEM((K, D), jnp.float32),
            pltpu.VMEM((K, D), jnp.float32),
            pltpu.VMEM_SHARED((SROWS, D), jnp.float32),
            pltpu.SemaphoreType.DMA,
            pltpu.SemaphoreType.DMA,
        ],
    )
    def seg_sum(h_hbm, src_hbm, dst_hbm, z_hbm, out_hbm,
                src_v, dst_v, buf0, buf1, agg_sh, sem0, sem1):
        c = lax.axis_index("c")
        s = lax.axis_index("s")

        pltpu.sync_copy(z_hbm, agg_sh.at[pl.ds(s * RPT, RPT)])
        pltpu.sync_copy(src_hbm.at[c, s], src_v)
        pltpu.sync_copy(dst_hbm.at[c, s], dst_v)
        plsc.subcore_barrier()

        def body(i, carry):
            c0 = 2 * i
            c1 = 2 * i + 1
            d0 = pltpu.async_copy(h_hbm.at[src_v.at[c0]], buf0, sem0)
            d1 = pltpu.async_copy(h_hbm.at[src_v.at[c1]], buf1, sem1)
            d0.wait()
            pltpu.sync_copy(buf0, agg_sh.at[dst_v.at[c0]], add=True)
            d1.wait()
            pltpu.sync_copy(buf1, agg_sh.at[dst_v.at[c1]], add=True)
            return carry
        lax.fori_loop(0, C2 // 2, body, 0)
        plsc.subcore_barrier()

        pltpu.sync_copy(agg_sh.at[pl.ds(s * RPT, RPT)],
                        out_hbm.at[pl.ds((c * NS + s) * RPT, RPT)])

    return seg_sum(h, src4, dst4, zrows)


def _sc_pool(h, row4, gdst4, zrows):
    mesh = plsc.VectorSubcoreMesh(core_axis_name="c", subcore_axis_name="s")
    BLK = 2 * GPT

    @functools.partial(
        pl.kernel,
        out_type=jax.ShapeDtypeStruct((NW, BLK, D), jnp.float32),
        mesh=mesh,
        scratch_types=[
            pltpu.VMEM((CG2, K), jnp.int32),
            pltpu.VMEM((CG2, K), jnp.int32),
            pltpu.VMEM((K, D), jnp.float32),
            pltpu.VMEM((K, D), jnp.float32),
            pltpu.VMEM_SHARED((NS * BLK, D), jnp.float32),
            pltpu.SemaphoreType.DMA,
            pltpu.SemaphoreType.DMA,
        ],
    )
    def pool(h_hbm, row_hbm, gdst_hbm, z_hbm, out_hbm,
             row_v, gdst_v, buf0, buf1, agg_sh, sem0, sem1):
        c = lax.axis_index("c")
        s = lax.axis_index("s")

        pltpu.sync_copy(z_hbm.at[pl.ds(0, BLK)],
                        agg_sh.at[pl.ds(s * BLK, BLK)])
        pltpu.sync_copy(row_hbm.at[c, s], row_v)
        pltpu.sync_copy(gdst_hbm.at[c, s], gdst_v)
        plsc.subcore_barrier()

        def body(i, carry):
            c0 = 2 * i
            c1 = 2 * i + 1
            d0 = pltpu.async_copy(h_hbm.at[row_v.at[c0]], buf0, sem0)
            d1 = pltpu.async_copy(h_hbm.at[row_v.at[c1]], buf1, sem1)
            d0.wait()
            pltpu.sync_copy(buf0, agg_sh.at[gdst_v.at[c0]], add=True)
            d1.wait()
            pltpu.sync_copy(buf1, agg_sh.at[gdst_v.at[c1]], add=True)
            return carry
        lax.fori_loop(0, CG2 // 2, body, 0)
        plsc.subcore_barrier()

        pltpu.sync_copy(agg_sh.at[pl.ds(s * BLK, BLK)],
                        out_hbm.at[c * NS + s])

    return pool(h, row4, gdst4, zrows)


def _mm_gin(h, agg, wt, b, e11):
    def body(h_ref, a_ref, w_ref, b_ref, e_ref, o_ref):
        xb = e_ref[0, 0] * h_ref[...] + a_ref[:N]
        o_ref[...] = jnp.dot(
            xb, w_ref[...], preferred_element_type=jnp.float32) + b_ref[...]

    return pl.pallas_call(
        body,
        out_shape=jax.ShapeDtypeStruct((N, wt.shape[1]), jnp.float32),
    )(h, agg, wt, b, e11)


def _mm_bias(x, wt, b):
    def body(x_ref, w_ref, b_ref, o_ref):
        o_ref[...] = jnp.dot(
            x_ref[...], w_ref[...],
            preferred_element_type=jnp.float32) + b_ref[...]

    return pl.pallas_call(
        body,
        out_shape=jax.ShapeDtypeStruct((x.shape[0], wt.shape[1]), jnp.float32),
    )(x, wt, b)


def _bn_act(x, g, b):
    m = jnp.mean(x, axis=0)
    v = jnp.var(x, axis=0)
    return jax.nn.relu(g * (x - m) / jnp.sqrt(v + 1e-5) + b)


def kernel(x, edge_index, batch, params, mlp_params):
    src = edge_index[0]
    dst = edge_index[1]

    owner = dst // RPT
    perm = jnp.argsort(owner, stable=True)
    packed = src * 16384 + dst
    packedp = packed[perm]
    owner_sorted = owner[perm]
    bids = jnp.arange(NW, dtype=jnp.int32)
    starts = jnp.searchsorted(owner_sorted, bids, side='left').astype(jnp.int32)
    ends = jnp.concatenate([starts[1:], jnp.array([E], jnp.int32)])
    slots = jnp.arange(CAP, dtype=jnp.int32)[None, :]
    idxm = starts[:, None] + slots
    valid = idxm < ends[:, None]
    idxc = jnp.minimum(idxm, E - 1)
    pp = packedp[idxc]
    ppsrc = pp // 16384
    ppdst = pp - ppsrc * 16384
    sc_of_b = (bids // NS)[:, None]
    psrc = jnp.where(valid, ppsrc, (idxm * 61) % N)
    pdst = jnp.where(valid, ppdst - sc_of_b * (NS * RPT),
                     NS * RPT + (idxm % 256))
    src4 = psrc.reshape(NC, NS, C2, K)
    dst4 = pdst.reshape(NC, NS, C2, K)

    gstarts = jnp.searchsorted(
        batch, (bids * GPT).astype(jnp.int32), side='left').astype(jnp.int32)
    gends = jnp.concatenate([gstarts[1:], jnp.array([N], jnp.int32)])
    gslots = jnp.arange(CAPG, dtype=jnp.int32)[None, :]
    gidx = gstarts[:, None] + gslots
    gvalid = gidx < gends[:, None]
    gidxc = jnp.minimum(gidx, N - 1)
    bloc = (bids % NS)[:, None] * (2 * GPT)
    prow = jnp.where(gvalid, gidxc, (gidx * 61) % N)
    pgdst = jnp.where(gvalid, batch[gidxc] - bids[:, None] * GPT + bloc,
                      bloc + GPT + (gidx % GPT))
    row4 = prow.reshape(NC, NS, CG2, K)
    gdst4 = pgdst.reshape(NC, NS, CG2, K)

    zrows = jnp.zeros((RPT, D), jnp.float32)

    h = x
    for p in params:
        agg = _sc_segment_sum(h, src4, dst4, zrows)
        z = _mm_gin(h, agg, p['W1'].T, p['b1'].reshape(1, -1),
                    (1.0 + p['eps']).reshape(1, 1))
        z = _bn_act(z, p['g1'], p['be1'])
        z = _mm_bias(z, p['W2'].T, p['b2'].reshape(1, -1))
        h = _bn_act(z, p['g2'], p['be2'])

    pooled = _sc_pool(h, row4, gdst4, zrows)[:, :GPT, :].reshape(G, D)
    z = _mm_bias(pooled, mlp_params['W1'].T, mlp_params['b1'].reshape(1, -1))
    z = _bn_act(z, mlp_params['g'], mlp_params['be'])
    return _mm_bias(z, mlp_params['W2'].T, mlp_params['b2'].reshape(1, -1))
